# baseline (speedup 1.0000x reference)
import functools

import jax
import jax.numpy as jnp
from jax import lax
from jax.experimental import pallas as pl
from jax.experimental.pallas import tpu as pltpu
from jax.experimental.pallas import tpu_sc as plsc

N = 10000
E = 320000
D_IN = 128
HID = 16
NCLS = 40

NC = 2
NS = 16
NW = NC * NS
CHUNK = 80
K = E // (NW * CHUNK)
NPAD = 10240
RPT = NPAD // NS
DEG_F = 8

FPAD = 128
W_DEG = 16


@functools.cache
def _make_deg():
    mesh = plsc.VectorSubcoreMesh(core_axis_name="c", subcore_axis_name="s")

    @functools.partial(
        pl.kernel,
        out_type=jax.ShapeDtypeStruct((NC, NPAD, FPAD), jnp.float32),
        mesh=mesh,
        scratch_types=[
            pltpu.VMEM((K, CHUNK), jnp.int32),
            pltpu.VMEM((CHUNK, DEG_F), jnp.float32),
            pltpu.VMEM_SHARED((NPAD, DEG_F), jnp.float32),
            pltpu.SemaphoreType.DMA,
        ],
        compiler_params=pltpu.CompilerParams(use_tc_tiling_on_sc=False),
    )
    def deg_kernel(dst_hbm, ones_hbm, zeros_hbm, out_hbm, dst_v, ones_v,
                   acc_sh, ssem):
        c = lax.axis_index("c")
        s = lax.axis_index("s")
        wid = s * NC + c
        pltpu.sync_copy(zeros_hbm.at[pl.ds(s * RPT, RPT)],
                        acc_sh.at[pl.ds(s * RPT, RPT)])
        pltpu.sync_copy(ones_hbm, ones_v)
        pltpu.sync_copy(dst_hbm.at[wid], dst_v)
        plsc.subcore_barrier()

        def body(j, carry):
            pltpu.async_copy(ones_v, acc_sh.at[dst_v.at[j]], ssem, add=True)

            @pl.when(j >= W_DEG)
            def _():
                pltpu.make_async_copy(
                    ones_v, acc_sh.at[dst_v.at[j - W_DEG]], ssem).wait()

            return carry

        lax.fori_loop(0, K, body, 0)
        for i in range(K - W_DEG, K):
            pltpu.make_async_copy(ones_v, acc_sh.at[dst_v.at[i]], ssem).wait()
        plsc.subcore_barrier()
        pltpu.sync_copy(acc_sh.at[pl.ds(s * RPT, RPT)],
                        out_hbm.at[c, pl.ds(s * RPT, RPT), pl.ds(0, DEG_F)])

    return deg_kernel


@functools.cache
def _make_agg(F, NB, DP, spmem_tab):
    mesh = plsc.VectorSubcoreMesh(core_axis_name="c", subcore_axis_name="s")

    scratch = [
        pltpu.VMEM((K, CHUNK), jnp.int32),
        pltpu.VMEM((K, CHUNK), jnp.int32),
        pltpu.VMEM((NB, CHUNK, F), jnp.float32),
        pltpu.VMEM_SHARED((NPAD, F), jnp.float32),
        pltpu.SemaphoreType.DMA,
        pltpu.SemaphoreType.DMA,
    ]
    if spmem_tab:
        scratch += [pltpu.VMEM((RPT, F), jnp.float32),
                    pltpu.VMEM_SHARED((NPAD, F), jnp.float32)]

    @functools.partial(
        pl.kernel,
        out_type=jax.ShapeDtypeStruct((NC, NPAD, FPAD), jnp.float32),
        mesh=mesh,
        scratch_types=scratch,
        compiler_params=pltpu.CompilerParams(use_tc_tiling_on_sc=False),
    )
    def agg_kernel(t_hbm, src_hbm, dst_hbm, zeros_hbm, out_hbm,
                   src_v, dst_v, rows_v, acc_sh, gsem, ssem,
                   t_v=None, tab_sh=None):
        c = lax.axis_index("c")
        s = lax.axis_index("s")
        wid = s * NC + c
        r0 = s * RPT
        pltpu.sync_copy(zeros_hbm.at[pl.ds(r0, RPT)],
                        acc_sh.at[pl.ds(r0, RPT)])
        pltpu.sync_copy(src_hbm.at[wid], src_v)
        pltpu.sync_copy(dst_hbm.at[wid], dst_v)
        if spmem_tab:
            pltpu.sync_copy(t_hbm.at[pl.ds(r0, RPT), pl.ds(0, F)], t_v)
            pltpu.sync_copy(t_v, tab_sh.at[pl.ds(r0, RPT)])
            tab = tab_sh
        else:
            tab = t_hbm
        plsc.subcore_barrier()

        for p in range(DP):
            pltpu.async_copy(tab.at[src_v.at[p]], rows_v.at[p], gsem)

        def body(j, carry):
            @pl.when(jnp.logical_and(j + DP >= NB, j + DP < K))
            def _():
                pltpu.make_async_copy(
                    rows_v.at[(j + DP) % NB],
                    acc_sh.at[dst_v.at[j + DP - NB]], ssem).wait()

            @pl.when(j + DP < K)
            def _():
                pltpu.async_copy(tab.at[src_v.at[j + DP]],
                                 rows_v.at[(j + DP) % NB], gsem)

            pltpu.make_async_copy(tab.at[src_v.at[j]],
                                  rows_v.at[j % NB], gsem).wait()
            pltpu.async_copy(rows_v.at[j % NB], acc_sh.at[dst_v.at[j]],
                             ssem, add=True)
            return carry

        lax.fori_loop(0, K, body, 0)
        for i in range(K - NB, K):
            pltpu.make_async_copy(rows_v.at[i % NB],
                                  acc_sh.at[dst_v.at[i]], ssem).wait()
        plsc.subcore_barrier()
        pltpu.sync_copy(acc_sh.at[pl.ds(r0, RPT)],
                        out_hbm.at[c, pl.ds(r0, RPT), pl.ds(0, F)])

    return agg_kernel


BLK = 2000
DIS_LANE = HID


def _tc_h1_body(x_ref, w1_ref, h1_ref):
    h1_ref[...] = jnp.dot(x_ref[...], w1_ref[...],
                          preferred_element_type=jnp.float32)


def _tc_1_body(h1_ref, d0_ref, d1_ref, p1_ref):
    deg = d0_ref[0, :, 0:1] + d1_ref[0, :, 0:1] + 1.0
    dis = lax.rsqrt(deg)
    p1_ref[:, :HID] = h1_ref[...] * dis
    p1_ref[:, DIS_LANE:DIS_LANE + 1] = dis


def _tc_2_body(a0_ref, a1_ref, p1_ref, b1_ref, w2_ref, t2c_ref):
    dis = p1_ref[:, DIS_LANE:DIS_LANE + 1]
    t1 = p1_ref[:, :HID]
    h = jnp.maximum((a0_ref[0, :, :HID] + a1_ref[0, :, :HID] + t1) * dis
                    + b1_ref[...], 0.0)
    t2c_ref[...] = jnp.dot(h, w2_ref[...],
                           preferred_element_type=jnp.float32) * dis


def _tc_3_body(b0_ref, b1_ref, p1_ref, t2c_ref, bias_ref, out_ref):
    dis = p1_ref[:, DIS_LANE:DIS_LANE + 1]
    z = ((b0_ref[0, :, :NCLS] + b1_ref[0, :, :NCLS] + t2c_ref[...]) * dis
         + bias_ref[...])
    m = jnp.max(z, axis=1, keepdims=True)
    lse = jnp.log(jnp.sum(jnp.exp(z - m), axis=1, keepdims=True)) + m
    out_ref[...] = z - lse


def _row_spec(f):
    return pl.BlockSpec((BLK, f), lambda i: (i, 0))


def _part_spec(cc, f):
    return pl.BlockSpec((1, BLK, f), lambda i, c=cc: (c, i, 0))


def _full_spec(r, cdim):
    return pl.BlockSpec((r, cdim), lambda i: (0, 0))


_tc_h1 = pl.pallas_call(
    _tc_h1_body,
    grid=(N // BLK,),
    in_specs=[_row_spec(D_IN), _full_spec(D_IN, HID)],
    out_specs=_row_spec(HID),
    out_shape=jax.ShapeDtypeStruct((N, HID), jnp.float32),
)

_tc_1 = pl.pallas_call(
    _tc_1_body,
    grid=(N // BLK,),
    in_specs=[_row_spec(HID), _part_spec(0, FPAD), _part_spec(1, FPAD)],
    out_specs=_row_spec(FPAD),
    out_shape=jax.ShapeDtypeStruct((NPAD, FPAD), jnp.float32),
)

_tc_2 = pl.pallas_call(
    _tc_2_body,
    grid=(N // BLK,),
    in_specs=[_part_spec(0, FPAD), _part_spec(1, FPAD), _row_spec(FPAD),
              _full_spec(1, HID), _full_spec(HID, NCLS)],
    out_specs=_row_spec(NCLS),
    out_shape=jax.ShapeDtypeStruct((N, NCLS), jnp.float32),
)

_tc_3 = pl.pallas_call(
    _tc_3_body,
    grid=(N // BLK,),
    in_specs=[_part_spec(0, FPAD), _part_spec(1, FPAD), _row_spec(FPAD),
              _row_spec(NCLS), _full_spec(1, NCLS)],
    out_specs=_row_spec(NCLS),
    out_shape=jax.ShapeDtypeStruct((N, NCLS), jnp.float32),
)


def kernel(x, edge_index, W1, b1, W2, b2):
    f32 = jnp.float32
    src = edge_index[0].reshape(NW, K, CHUNK)
    dst = edge_index[1].reshape(NW, K, CHUNK)

    ones_blk = jnp.ones((CHUNK, DEG_F), f32)
    zeros_deg = jnp.zeros((NPAD, DEG_F), f32)
    zeros16 = jnp.zeros((NPAD, HID), f32)
    zeros40 = jnp.zeros((NPAD, NCLS), f32)

    deg_parts = _make_deg()(dst, ones_blk, zeros_deg)
    h1 = _tc_h1(x, W1)
    p1 = _tc_1(h1, deg_parts, deg_parts)

    agg1 = _make_agg(HID, 28, 14, True)(p1, src, dst, zeros16)
    t2c = _tc_2(agg1, agg1, p1, b1.reshape(1, HID), W2)

    agg2 = _make_agg(NCLS, 22, 11, False)(t2c, src, dst, zeros40)
    return _tc_3(agg2, agg2, p1, t2c, b2.reshape(1, NCLS))


# speedup vs baseline: 1.0272x; 1.0272x over previous
import functools

import jax
import jax.numpy as jnp
from jax import lax
from jax.experimental import pallas as pl
from jax.experimental.pallas import tpu as pltpu
from jax.experimental.pallas import tpu_sc as plsc

N = 10000
E = 320000
D_IN = 128
HID = 16
NCLS = 40

NC = 2
NS = 16
NW = NC * NS
CHUNK = 80
K = E // (NW * CHUNK)
NPAD = 10240
RPT = NPAD // NS
DEG_F = 8

FPAD = 128
W_DEG = 16


@functools.cache
def _make_deg():
    mesh = plsc.VectorSubcoreMesh(core_axis_name="c", subcore_axis_name="s")

    @functools.partial(
        pl.kernel,
        out_type=jax.ShapeDtypeStruct((NC, NPAD, FPAD), jnp.float32),
        mesh=mesh,
        scratch_types=[
            pltpu.VMEM((K, CHUNK), jnp.int32),
            pltpu.VMEM((CHUNK, DEG_F), jnp.float32),
            pltpu.VMEM_SHARED((NPAD, DEG_F), jnp.float32),
            pltpu.SemaphoreType.DMA,
        ],
        compiler_params=pltpu.CompilerParams(use_tc_tiling_on_sc=False),
    )
    def deg_kernel(dst_hbm, ones_hbm, zeros_hbm, out_hbm, dst_v, ones_v,
                   acc_sh, ssem):
        c = lax.axis_index("c")
        s = lax.axis_index("s")
        wid = s * NC + c
        pltpu.sync_copy(zeros_hbm.at[pl.ds(s * RPT, RPT)],
                        acc_sh.at[pl.ds(s * RPT, RPT)])
        pltpu.sync_copy(ones_hbm, ones_v)
        pltpu.sync_copy(dst_hbm.at[wid], dst_v)
        plsc.subcore_barrier()

        def body(j, carry):
            pltpu.async_copy(ones_v, acc_sh.at[dst_v.at[j]], ssem, add=True)

            @pl.when(j >= W_DEG)
            def _():
                pltpu.make_async_copy(
                    ones_v, acc_sh.at[dst_v.at[j - W_DEG]], ssem).wait()

            return carry

        lax.fori_loop(0, K, body, 0)
        for i in range(K - W_DEG, K):
            pltpu.make_async_copy(ones_v, acc_sh.at[dst_v.at[i]], ssem).wait()
        plsc.subcore_barrier()
        pltpu.sync_copy(acc_sh.at[pl.ds(s * RPT, RPT)],
                        out_hbm.at[c, pl.ds(s * RPT, RPT), pl.ds(0, DEG_F)])

    return deg_kernel


@functools.cache
def _make_agg(F, NB, DP, spmem_tab):
    mesh = plsc.VectorSubcoreMesh(core_axis_name="c", subcore_axis_name="s")

    scratch = [
        pltpu.VMEM((K, CHUNK), jnp.int32),
        pltpu.VMEM((K, CHUNK), jnp.int32),
        pltpu.VMEM((NB, CHUNK, F), jnp.float32),
        pltpu.VMEM_SHARED((NPAD, F), jnp.float32),
        pltpu.SemaphoreType.DMA,
        pltpu.SemaphoreType.DMA,
    ]
    if spmem_tab:
        scratch += [pltpu.VMEM((RPT, F), jnp.float32),
                    pltpu.VMEM_SHARED((NPAD, F), jnp.float32)]

    @functools.partial(
        pl.kernel,
        out_type=jax.ShapeDtypeStruct((NC, NPAD, FPAD), jnp.float32),
        mesh=mesh,
        scratch_types=scratch,
        compiler_params=pltpu.CompilerParams(use_tc_tiling_on_sc=False),
    )
    def agg_kernel(t_hbm, src_hbm, dst_hbm, zeros_hbm, out_hbm,
                   src_v, dst_v, rows_v, acc_sh, gsem, ssem,
                   t_v=None, tab_sh=None):
        c = lax.axis_index("c")
        s = lax.axis_index("s")
        wid = s * NC + c
        r0 = s * RPT
        pltpu.sync_copy(zeros_hbm.at[pl.ds(r0, RPT)],
                        acc_sh.at[pl.ds(r0, RPT)])
        pltpu.sync_copy(src_hbm.at[wid], src_v)
        pltpu.sync_copy(dst_hbm.at[wid], dst_v)
        if spmem_tab:
            pltpu.sync_copy(t_hbm.at[pl.ds(r0, RPT), pl.ds(0, F)], t_v)
            pltpu.sync_copy(t_v, tab_sh.at[pl.ds(r0, RPT)])
            tab = tab_sh
        else:
            tab = t_hbm
        plsc.subcore_barrier()

        for p in range(DP):
            pltpu.async_copy(tab.at[src_v.at[p]], rows_v.at[p], gsem)

        def body(j, carry):
            @pl.when(jnp.logical_and(j + DP >= NB, j + DP < K))
            def _():
                pltpu.make_async_copy(
                    rows_v.at[(j + DP) % NB],
                    acc_sh.at[dst_v.at[j + DP - NB]], ssem).wait()

            @pl.when(j + DP < K)
            def _():
                pltpu.async_copy(tab.at[src_v.at[j + DP]],
                                 rows_v.at[(j + DP) % NB], gsem)

            pltpu.make_async_copy(tab.at[src_v.at[j]],
                                  rows_v.at[j % NB], gsem).wait()
            pltpu.async_copy(rows_v.at[j % NB], acc_sh.at[dst_v.at[j]],
                             ssem, add=True)
            return carry

        lax.fori_loop(0, K, body, 0)
        for i in range(K - NB, K):
            pltpu.make_async_copy(rows_v.at[i % NB],
                                  acc_sh.at[dst_v.at[i]], ssem).wait()
        plsc.subcore_barrier()
        pltpu.sync_copy(acc_sh.at[pl.ds(r0, RPT)],
                        out_hbm.at[c, pl.ds(r0, RPT), pl.ds(0, F)])

    return agg_kernel


BLK = 5000
DIS_LANE = HID


def _tc_h1_body(x_ref, w1_ref, h1_ref):
    h1_ref[...] = jnp.dot(x_ref[...], w1_ref[...],
                          preferred_element_type=jnp.float32)


def _tc_1_body(h1_ref, d0_ref, d1_ref, p1_ref):
    deg = d0_ref[0, :, 0:1] + d1_ref[0, :, 0:1] + 1.0
    dis = lax.rsqrt(deg)
    p1_ref[:, :HID] = h1_ref[...] * dis
    p1_ref[:, DIS_LANE:DIS_LANE + 1] = dis


def _tc_2_body(a0_ref, a1_ref, p1_ref, b1_ref, w2_ref, t2c_ref):
    dis = p1_ref[:, DIS_LANE:DIS_LANE + 1]
    t1 = p1_ref[:, :HID]
    h = jnp.maximum((a0_ref[0, :, :HID] + a1_ref[0, :, :HID] + t1) * dis
                    + b1_ref[...], 0.0)
    t2c_ref[...] = jnp.dot(h, w2_ref[...],
                           preferred_element_type=jnp.float32) * dis


def _tc_3_body(b0_ref, b1_ref, p1_ref, t2c_ref, bias_ref, out_ref):
    dis = p1_ref[:, DIS_LANE:DIS_LANE + 1]
    z = ((b0_ref[0, :, :NCLS] + b1_ref[0, :, :NCLS] + t2c_ref[...]) * dis
         + bias_ref[...])
    m = jnp.max(z, axis=1, keepdims=True)
    lse = jnp.log(jnp.sum(jnp.exp(z - m), axis=1, keepdims=True)) + m
    out_ref[...] = z - lse


def _row_spec(f):
    return pl.BlockSpec((BLK, f), lambda i: (i, 0))


def _part_spec(cc, f):
    return pl.BlockSpec((1, BLK, f), lambda i, c=cc: (c, i, 0))


def _full_spec(r, cdim):
    return pl.BlockSpec((r, cdim), lambda i: (0, 0))


_tc_h1 = pl.pallas_call(
    _tc_h1_body,
    grid=(N // BLK,),
    in_specs=[_row_spec(D_IN), _full_spec(D_IN, HID)],
    out_specs=_row_spec(HID),
    out_shape=jax.ShapeDtypeStruct((N, HID), jnp.float32),
)

_tc_1 = pl.pallas_call(
    _tc_1_body,
    grid=(N // BLK,),
    in_specs=[_row_spec(HID), _part_spec(0, FPAD), _part_spec(1, FPAD)],
    out_specs=_row_spec(FPAD),
    out_shape=jax.ShapeDtypeStruct((NPAD, FPAD), jnp.float32),
)

_tc_2 = pl.pallas_call(
    _tc_2_body,
    grid=(N // BLK,),
    in_specs=[_part_spec(0, FPAD), _part_spec(1, FPAD), _row_spec(FPAD),
              _full_spec(1, HID), _full_spec(HID, NCLS)],
    out_specs=_row_spec(NCLS),
    out_shape=jax.ShapeDtypeStruct((N, NCLS), jnp.float32),
)

_tc_3 = pl.pallas_call(
    _tc_3_body,
    grid=(N // BLK,),
    in_specs=[_part_spec(0, FPAD), _part_spec(1, FPAD), _row_spec(FPAD),
              _row_spec(NCLS), _full_spec(1, NCLS)],
    out_specs=_row_spec(NCLS),
    out_shape=jax.ShapeDtypeStruct((N, NCLS), jnp.float32),
)


def kernel(x, edge_index, W1, b1, W2, b2):
    f32 = jnp.float32
    src = edge_index[0].reshape(NW, K, CHUNK)
    dst = edge_index[1].reshape(NW, K, CHUNK)

    ones_blk = jnp.ones((CHUNK, DEG_F), f32)
    zeros_deg = jnp.zeros((NPAD, DEG_F), f32)
    zeros16 = jnp.zeros((NPAD, HID), f32)
    zeros40 = jnp.zeros((NPAD, NCLS), f32)

    deg_parts = _make_deg()(dst, ones_blk, zeros_deg)
    h1 = _tc_h1(x, W1)
    p1 = _tc_1(h1, deg_parts, deg_parts)

    agg1 = _make_agg(HID, 28, 14, True)(p1, src, dst, zeros16)
    t2c = _tc_2(agg1, agg1, p1, b1.reshape(1, HID), W2)

    agg2 = _make_agg(NCLS, 22, 11, False)(t2c, src, dst, zeros40)
    return _tc_3(agg2, agg2, p1, t2c, b2.reshape(1, NCLS))
